# baseline (speedup 1.0000x reference)
import functools

import jax
import jax.numpy as jnp
from jax import lax
from jax.experimental import pallas as pl
from jax.experimental.pallas import tpu as pltpu
from jax.experimental.pallas import tpu_sc as plsc

B, F, V, D = 16384, 26, 1000000, 16
OUT_W = F * D + 1
L = 16
RPS = 128 // D

NC, NS = 2, 16
NW = NC * NS
ROWS_PER_TILE = B // NW
CB = 32
NCHUNK = ROWS_PER_TILE // CB

NVT = (V + 127) // 128
NVT_FULL = V // 128
VTAIL = V - NVT_FULL * 128
TPW = NVT_FULL // NW - (NVT_FULL // NW) % 2
NREST = NVT - TPW * NW
SUP = 4
NSUP = TPW // SUP
NBUF = 4


def _xpose_tile(tin, tout, hbase, k, ngroups):
    for d in range(D):
        for g in range(ngroups):
            v16 = tin[d, pl.ds(128 * k + g * L, L)]
            plsc.store_scatter(tout, [hbase + (2048 * k + 256 * g + d)], v16)


def _xpose_body(tok_t_hbm, tail_hbm, out_hbm,
                tin0, tin1, tin2, tin3, tout0, tout1, tout2, tout3,
                sem_i0, sem_i1, sem_i2, sem_i3,
                sem_o0, sem_o1, sem_o2, sem_o3):
    wid = lax.axis_index("s") * NC + lax.axis_index("c")
    tbase = wid * TPW
    riota = lax.iota(jnp.int32, L)
    hbase = (riota // 8) * 128 + (riota % 8) * D
    tins = (tin0, tin1, tin2, tin3)
    touts = (tout0, tout1, tout2, tout3)
    sem_is = (sem_i0, sem_i1, sem_i2, sem_i3)
    sem_os = (sem_o0, sem_o1, sem_o2, sem_o3)

    def in_src(t):
        return tok_t_hbm.at[:, pl.ds(t * 128, 128)]

    def sup_src(u):
        return tok_t_hbm.at[:, pl.ds((tbase + SUP * u) * 128, SUP * 128)]

    def sup_dst(u):
        return out_hbm.at[pl.ds((tbase + SUP * u) * 2048, SUP * 2048)]

    def sup_xpose(tin, tout):
        for k in range(SUP):
            _xpose_tile(tin, tout, hbase, k, 8)

    for p in range(NBUF):
        pltpu.async_copy(sup_src(p), tins[p], sem_is[p])

    def step(i, carry):
        for bidx in range(NBUF):
            u = NBUF * i + bidx
            tin, tout = tins[bidx], touts[bidx]
            sem_i, sem_o = sem_is[bidx], sem_os[bidx]
            pltpu.make_async_copy(sup_src(u), tin, sem_i).wait()

            @pl.when(u >= NBUF)
            def _drain():
                pltpu.make_async_copy(tout, sup_dst(u), sem_o).wait()

            sup_xpose(tin, tout)
            pltpu.async_copy(tout, sup_dst(u), sem_o)

            @pl.when(u + NBUF < NSUP)
            def _next():
                pltpu.async_copy(sup_src(u + NBUF), tin, sem_i)

        return carry

    lax.fori_loop(0, NSUP // NBUF, step, 0, unroll=False)
    u_last = NSUP - 1
    pltpu.make_async_copy(sup_src(u_last), tin0, sem_i0).wait()
    pltpu.make_async_copy(tout0, sup_dst(u_last), sem_o0).wait()
    sup_xpose(tin0, tout0)
    pltpu.sync_copy(tout0, sup_dst(u_last))
    for p in range(1, NBUF):
        pltpu.make_async_copy(touts[p], sup_dst(u_last - NBUF + p),
                              sem_os[p]).wait()

    @pl.when(wid < NREST - 1)
    def _rest_full():
        t = NW * TPW + wid
        pltpu.sync_copy(in_src(t), tin0.at[:, pl.ds(0, 128)])
        _xpose_tile(tin0, tout0, hbase, 0, 8)
        pltpu.sync_copy(tout0.at[pl.ds(0, 2048)],
                        out_hbm.at[pl.ds(t * 2048, 2048)])

    @pl.when(wid == NREST - 1)
    def _rest_tail():
        pltpu.sync_copy(tail_hbm, tin0.at[:, pl.ds(0, 128)])
        _xpose_tile(tin0, tout0, hbase, 0, VTAIL // L)
        pltpu.sync_copy(tout0.at[pl.ds(0, VTAIL * D)],
                        out_hbm.at[pl.ds(NVT_FULL * 2048, VTAIL * D)])


def _gather_body(idx_hbm, tok_hbm, fo_hbm, bias_hbm, out_hbm,
                 idx_v0, idx_v1, rows_v0, rows_v1, fo_v0, fo_v1,
                 out_v0, out_v1, bias_v,
                 sem_t0, sem_t1, sem_f0, sem_f1, sem_o0, sem_o1):
    wid = lax.axis_index("s") * NC + lax.axis_index("c")
    tile_base = wid * ROWS_PER_TILE
    pltpu.sync_copy(bias_hbm, bias_v)
    bias_vec = bias_v[...]
    riota = lax.iota(jnp.int32, L)
    idxs = (idx_v0, idx_v1)
    rows = (rows_v0, rows_v1)
    fos = (fo_v0, fo_v1)
    outs = (out_v0, out_v1)
    sem_ts = (sem_t0, sem_t1)
    sem_fs = (sem_f0, sem_f1)
    sem_os = (sem_o0, sem_o1)

    def stage(c, bidx):
        base = tile_base + c * CB
        pltpu.sync_copy(idx_hbm.at[pl.ds(base * F, CB * F)], idxs[bidx])
        pltpu.async_copy(tok_hbm.at[idxs[bidx]], rows[bidx], sem_ts[bidx])
        pltpu.async_copy(fo_hbm.at[idxs[bidx]], fos[bidx], sem_fs[bidx])

    stage(0, 0)
    stage(1, 1)

    def chunk(i, carry):
        for bidx in range(2):
            c = 2 * i + bidx
            base = tile_base + c * CB
            idx_v, rows_v, fo_v, out_v = (
                idxs[bidx], rows[bidx], fos[bidx], outs[bidx])
            pltpu.make_async_copy(tok_hbm.at[idx_v], rows_v, sem_ts[bidx]).wait()
            pltpu.make_async_copy(fo_hbm.at[idx_v], fo_v, sem_fs[bidx]).wait()

            @pl.when(c >= 2)
            def _drain():
                pltpu.make_async_copy(
                    out_v, out_hbm.at[pl.ds(base, CB)], sem_os[bidx]).wait()

            def row(b, carry2):
                for f in range(F):
                    out_v[b, pl.ds(f * D, D)] = rows_v[b * F + f]
                return carry2

            lax.fori_loop(0, CB, row, 0, unroll=False)

            def grp(g, carry2):
                b0 = g * L
                acc = bias_vec
                for f in range(F):
                    acc = acc + plsc.load_gather(fo_v, [(b0 + riota) * F + f])
                plsc.store_scatter(
                    out_v, [b0 + riota, jnp.full((L,), F * D, jnp.int32)], acc)
                return carry2

            lax.fori_loop(0, CB // L, grp, 0, unroll=False)

            pltpu.async_copy(out_v, out_hbm.at[pl.ds(base, CB)], sem_os[bidx])

            @pl.when(c + 2 < NCHUNK)
            def _next():
                stage(c + 2, bidx)

        return carry

    lax.fori_loop(0, NCHUNK // 2, chunk, 0, unroll=False)
    last0 = tile_base + (NCHUNK - 2) * CB
    last1 = tile_base + (NCHUNK - 1) * CB
    pltpu.make_async_copy(out_v0, out_hbm.at[pl.ds(last0, CB)], sem_o0).wait()
    pltpu.make_async_copy(out_v1, out_hbm.at[pl.ds(last1, CB)], sem_o1).wait()


@jax.jit
def _run(idx_flat, tok_t, tok_tail, fo_flat, bias16):
    mesh = plsc.VectorSubcoreMesh(core_axis_name="c", subcore_axis_name="s",
                                  num_cores=NC, num_subcores=NS)
    xpose = functools.partial(
        pl.kernel,
        mesh=mesh,
        out_type=jax.ShapeDtypeStruct((V * D,), jnp.float32),
        scratch_types=(
            [pltpu.VMEM((D, SUP * 128), jnp.float32)] * NBUF
            + [pltpu.VMEM((SUP * 2048,), jnp.float32)] * NBUF
            + [pltpu.SemaphoreType.DMA] * (2 * NBUF)
        ),
        compiler_params=pltpu.CompilerParams(
            needs_layout_passes=False, use_tc_tiling_on_sc=True),
    )(_xpose_body)
    tok_rm = xpose(tok_t, tok_tail).reshape(V, D)

    k = functools.partial(
        pl.kernel,
        mesh=mesh,
        out_type=jax.ShapeDtypeStruct((B, OUT_W), jnp.float32),
        scratch_types=[
            pltpu.VMEM((CB * F,), jnp.int32),
            pltpu.VMEM((CB * F,), jnp.int32),
            pltpu.VMEM((CB * F, D), jnp.float32),
            pltpu.VMEM((CB * F, D), jnp.float32),
            pltpu.VMEM((CB * F,), jnp.float32),
            pltpu.VMEM((CB * F,), jnp.float32),
            pltpu.VMEM((CB, OUT_W), jnp.float32),
            pltpu.VMEM((CB, OUT_W), jnp.float32),
            pltpu.VMEM((L,), jnp.float32),
            pltpu.SemaphoreType.DMA,
            pltpu.SemaphoreType.DMA,
            pltpu.SemaphoreType.DMA,
            pltpu.SemaphoreType.DMA,
            pltpu.SemaphoreType.DMA,
            pltpu.SemaphoreType.DMA,
        ],
        compiler_params=pltpu.CompilerParams(
            needs_layout_passes=False, use_tc_tiling_on_sc=False),
    )(_gather_body)
    return k(idx_flat, tok_rm, fo_flat, bias16)


def kernel(indices, token_table, first_order_table, first_order_bias):
    idx_flat = indices.reshape(-1)
    tok_t = token_table.T
    tok_tail = jnp.pad(token_table[V - VTAIL:].T, ((0, 0), (0, 128 - VTAIL)))
    fo_flat = first_order_table.reshape(-1)
    bias16 = jnp.broadcast_to(first_order_bias, (L,))
    return _run(idx_flat, tok_t, tok_tail, fo_flat, bias16)


# speedup vs baseline: 1.0069x; 1.0069x over previous
import functools

import jax
import jax.numpy as jnp
from jax import lax
from jax.experimental import pallas as pl
from jax.experimental.pallas import tpu as pltpu
from jax.experimental.pallas import tpu_sc as plsc

B, F, V, D = 16384, 26, 1000000, 16
OUT_W = F * D + 1
L = 16
RPS = 128 // D

NC, NS = 2, 16
NW = NC * NS
ROWS_PER_TILE = B // NW
CB = 32
NCHUNK = ROWS_PER_TILE // CB

NVT = (V + 127) // 128
NVT_FULL = V // 128
VTAIL = V - NVT_FULL * 128
TPW = NVT_FULL // NW - (NVT_FULL // NW) % 2
NREST = NVT - TPW * NW
SUP = 4
NSUP = TPW // SUP


def _xpose_tile(tin, tout, hbase, k, ngroups):
    for d in range(D):
        for g in range(ngroups):
            v16 = tin[d, pl.ds(128 * k + g * L, L)]
            plsc.store_scatter(tout, [hbase + (2048 * k + 256 * g + d)], v16)


def _xpose_body(tok_t_hbm, tail_hbm, out_hbm,
                tin0, tin1, tout0, tout1,
                sem_i0, sem_i1, sem_o0, sem_o1):
    wid = lax.axis_index("s") * NC + lax.axis_index("c")
    tbase = wid * TPW
    riota = lax.iota(jnp.int32, L)
    hbase = (riota // 8) * 128 + (riota % 8) * D
    tins = (tin0, tin1)
    touts = (tout0, tout1)
    sem_is = (sem_i0, sem_i1)
    sem_os = (sem_o0, sem_o1)

    def in_src(t):
        return tok_t_hbm.at[:, pl.ds(t * 128, 128)]

    def sup_src(u):
        return tok_t_hbm.at[:, pl.ds((tbase + SUP * u) * 128, SUP * 128)]

    def sup_dst(u):
        return out_hbm.at[pl.ds((tbase + SUP * u) * 2048, SUP * 2048)]

    def sup_xpose(tin, tout):
        for k in range(SUP):
            _xpose_tile(tin, tout, hbase, k, 8)

    pltpu.async_copy(sup_src(0), tin0, sem_i0)
    pltpu.async_copy(sup_src(1), tin1, sem_i1)

    def step(i, carry):
        for bidx in range(2):
            u = 2 * i + bidx
            tin, tout = tins[bidx], touts[bidx]
            sem_i, sem_o = sem_is[bidx], sem_os[bidx]
            pltpu.make_async_copy(sup_src(u), tin, sem_i).wait()

            @pl.when(u >= 2)
            def _drain():
                pltpu.make_async_copy(tout, sup_dst(u), sem_o).wait()

            sup_xpose(tin, tout)
            pltpu.async_copy(tout, sup_dst(u), sem_o)

            @pl.when(u + 2 < NSUP)
            def _next():
                pltpu.async_copy(sup_src(u + 2), tin, sem_i)

        return carry

    lax.fori_loop(0, NSUP // 2, step, 0, unroll=False)
    u_last = NSUP - 1
    pltpu.make_async_copy(sup_src(u_last), tin0, sem_i0).wait()
    pltpu.make_async_copy(tout0, sup_dst(u_last), sem_o0).wait()
    sup_xpose(tin0, tout0)
    pltpu.async_copy(tout0, sup_dst(u_last), sem_o0)
    pltpu.make_async_copy(tout0, sup_dst(u_last), sem_o0).wait()
    pltpu.make_async_copy(tout1, sup_dst(u_last - 1), sem_o1).wait()

    @pl.when(wid < NREST - 1)
    def _rest_full():
        t = NW * TPW + wid
        pltpu.sync_copy(in_src(t), tin0.at[:, pl.ds(0, 128)])
        _xpose_tile(tin0, tout0, hbase, 0, 8)
        pltpu.sync_copy(tout0.at[pl.ds(0, 2048)],
                        out_hbm.at[pl.ds(t * 2048, 2048)])

    @pl.when(wid == NREST - 1)
    def _rest_tail():
        pltpu.sync_copy(tail_hbm, tin0.at[:, pl.ds(0, 128)])
        _xpose_tile(tin0, tout0, hbase, 0, VTAIL // L)
        pltpu.sync_copy(tout0.at[pl.ds(0, VTAIL * D)],
                        out_hbm.at[pl.ds(NVT_FULL * 2048, VTAIL * D)])


def _gather_body(idx_hbm, tok_hbm, fo_hbm, bias_hbm, out_hbm,
                 idx_v0, idx_v1, rows_v0, rows_v1, fo_v0, fo_v1,
                 out_v0, out_v1, bias_v,
                 sem_t0, sem_t1, sem_f0, sem_f1, sem_o0, sem_o1):
    wid = lax.axis_index("s") * NC + lax.axis_index("c")
    tile_base = wid * ROWS_PER_TILE
    pltpu.sync_copy(bias_hbm, bias_v)
    bias_vec = bias_v[...]
    riota = lax.iota(jnp.int32, L)
    idxs = (idx_v0, idx_v1)
    rows = (rows_v0, rows_v1)
    fos = (fo_v0, fo_v1)
    outs = (out_v0, out_v1)
    sem_ts = (sem_t0, sem_t1)
    sem_fs = (sem_f0, sem_f1)
    sem_os = (sem_o0, sem_o1)

    def stage(c, bidx):
        base = tile_base + c * CB
        pltpu.sync_copy(idx_hbm.at[pl.ds(base * F, CB * F)], idxs[bidx])
        pltpu.async_copy(tok_hbm.at[idxs[bidx]], rows[bidx], sem_ts[bidx])
        pltpu.async_copy(fo_hbm.at[idxs[bidx]], fos[bidx], sem_fs[bidx])

    stage(0, 0)
    stage(1, 1)

    def chunk(i, carry):
        for bidx in range(2):
            c = 2 * i + bidx
            base = tile_base + c * CB
            idx_v, rows_v, fo_v, out_v = (
                idxs[bidx], rows[bidx], fos[bidx], outs[bidx])
            pltpu.make_async_copy(tok_hbm.at[idx_v], rows_v, sem_ts[bidx]).wait()
            pltpu.make_async_copy(fo_hbm.at[idx_v], fo_v, sem_fs[bidx]).wait()

            @pl.when(c >= 2)
            def _drain():
                pltpu.make_async_copy(
                    out_v, out_hbm.at[pl.ds(base, CB)], sem_os[bidx]).wait()

            def row(b, carry2):
                for f in range(F):
                    out_v[b, pl.ds(f * D, D)] = rows_v[b * F + f]
                return carry2

            lax.fori_loop(0, CB, row, 0, unroll=False)

            def grp(g, carry2):
                b0 = g * L
                acc = bias_vec
                for f in range(F):
                    acc = acc + plsc.load_gather(fo_v, [(b0 + riota) * F + f])
                plsc.store_scatter(
                    out_v, [b0 + riota, jnp.full((L,), F * D, jnp.int32)], acc)
                return carry2

            lax.fori_loop(0, CB // L, grp, 0, unroll=False)

            pltpu.async_copy(out_v, out_hbm.at[pl.ds(base, CB)], sem_os[bidx])

            @pl.when(c + 2 < NCHUNK)
            def _next():
                stage(c + 2, bidx)

        return carry

    lax.fori_loop(0, NCHUNK // 2, chunk, 0, unroll=False)
    last0 = tile_base + (NCHUNK - 2) * CB
    last1 = tile_base + (NCHUNK - 1) * CB
    pltpu.make_async_copy(out_v0, out_hbm.at[pl.ds(last0, CB)], sem_o0).wait()
    pltpu.make_async_copy(out_v1, out_hbm.at[pl.ds(last1, CB)], sem_o1).wait()


@jax.jit
def _run(idx_flat, tok_t, tok_tail, fo_flat, bias16):
    mesh = plsc.VectorSubcoreMesh(core_axis_name="c", subcore_axis_name="s",
                                  num_cores=NC, num_subcores=NS)
    xpose = functools.partial(
        pl.kernel,
        mesh=mesh,
        out_type=jax.ShapeDtypeStruct((V * D,), jnp.float32),
        scratch_types=[
            pltpu.VMEM((D, SUP * 128), jnp.float32),
            pltpu.VMEM((D, SUP * 128), jnp.float32),
            pltpu.VMEM((SUP * 2048,), jnp.float32),
            pltpu.VMEM((SUP * 2048,), jnp.float32),
            pltpu.SemaphoreType.DMA,
            pltpu.SemaphoreType.DMA,
            pltpu.SemaphoreType.DMA,
            pltpu.SemaphoreType.DMA,
        ],
        compiler_params=pltpu.CompilerParams(
            needs_layout_passes=False, use_tc_tiling_on_sc=True),
    )(_xpose_body)
    tok_rm = xpose(tok_t, tok_tail).reshape(V, D)

    k = functools.partial(
        pl.kernel,
        mesh=mesh,
        out_type=jax.ShapeDtypeStruct((B, OUT_W), jnp.float32),
        scratch_types=[
            pltpu.VMEM((CB * F,), jnp.int32),
            pltpu.VMEM((CB * F,), jnp.int32),
            pltpu.VMEM((CB * F, D), jnp.float32),
            pltpu.VMEM((CB * F, D), jnp.float32),
            pltpu.VMEM((CB * F,), jnp.float32),
            pltpu.VMEM((CB * F,), jnp.float32),
            pltpu.VMEM((CB, OUT_W), jnp.float32),
            pltpu.VMEM((CB, OUT_W), jnp.float32),
            pltpu.VMEM((L,), jnp.float32),
            pltpu.SemaphoreType.DMA,
            pltpu.SemaphoreType.DMA,
            pltpu.SemaphoreType.DMA,
            pltpu.SemaphoreType.DMA,
            pltpu.SemaphoreType.DMA,
            pltpu.SemaphoreType.DMA,
        ],
        compiler_params=pltpu.CompilerParams(
            needs_layout_passes=False, use_tc_tiling_on_sc=False),
    )(_gather_body)
    return k(idx_flat, tok_rm, fo_flat, bias16)


def kernel(indices, token_table, first_order_table, first_order_bias):
    idx_flat = indices.reshape(-1)
    tok_t = token_table.T
    tok_tail = jnp.pad(token_table[V - VTAIL:].T, ((0, 0), (0, 128 - VTAIL)))
    fo_flat = first_order_table.reshape(-1)
    bias16 = jnp.broadcast_to(first_order_bias, (L,))
    return _run(idx_flat, tok_t, tok_tail, fo_flat, bias16)
